# baseline (speedup 1.0000x reference)
import functools

import jax
import jax.numpy as jnp
from jax import lax
from jax.experimental import pallas as pl
from jax.experimental.pallas import tpu as pltpu
from jax.experimental.pallas import tpu_sc as plsc

_B, _D, _C = 16384, 128, 100
_MOM = 0.9

_NC, _NS = 2, 16
_NW = _NC * _NS
_RPW = _B // _NW
_CHUNK = 128
_NCHUNK = _RPW // _CHUNK
_CPAD = 128
_ZROWS = _CPAD // _NS


def _sc_segsum_body(emb_hbm, lab_hbm, zeros_hbm, psums_hbm,
                    emb_v0, emb_v1, emb_v2, emb_v3,
                    idx_v0, idx_v1, idx_v2, idx_v3, sums_sh,
                    sem0, sem1, sem2, sem3, ssem):
    cid = lax.axis_index("c")
    sid = lax.axis_index("s")
    base = (cid * _NS + sid) * _RPW

    embs = (emb_v0, emb_v1, emb_v2, emb_v3)
    idxs = (idx_v0, idx_v1, idx_v2, idx_v3)
    sems = (sem0, sem1, sem2, sem3)

    descs = []
    for k in range(_NCHUNK):
        off = base + k * _CHUNK
        descs.append((
            pltpu.async_copy(lab_hbm.at[pl.ds(off, _CHUNK)], idxs[k], sems[k]),
            pltpu.async_copy(emb_hbm.at[pl.ds(off, _CHUNK)], embs[k], sems[k]),
        ))

    pltpu.sync_copy(zeros_hbm, sums_sh.at[pl.ds(sid * _ZROWS, _ZROWS)])
    plsc.subcore_barrier()

    scat = []
    for k in range(_NCHUNK):
        d1, d2 = descs[k]
        d1.wait()
        d2.wait()
        scat.append(pltpu.async_copy(embs[k], sums_sh.at[idxs[k]], ssem,
                                     add=True))
    for d in scat:
        d.wait()
    plsc.subcore_barrier()

    r0 = sid * _ZROWS
    pltpu.sync_copy(sums_sh.at[pl.ds(r0, _ZROWS)],
                    psums_hbm.at[cid, pl.ds(r0, _ZROWS)])


@functools.cache
def _sc_segsum():
    return pl.kernel(
        _sc_segsum_body,
        out_type=jax.ShapeDtypeStruct((_NC, _CPAD, _D), jnp.float32),
        mesh=plsc.VectorSubcoreMesh(core_axis_name="c", subcore_axis_name="s",
                                    num_cores=_NC, num_subcores=_NS),
        scratch_types=(
            [pltpu.VMEM((_CHUNK, _D), jnp.float32)] * _NCHUNK
            + [pltpu.VMEM((_CHUNK,), jnp.int32)] * _NCHUNK
            + [pltpu.VMEM_SHARED((_CPAD, _D), jnp.float32)]
            + [pltpu.SemaphoreType.DMA] * (_NCHUNK + 1)
        ),
    )


_BLK = 2048
_NBLK = _B // _BLK
_MBLK = 4096
_NMBLK = _B // _MBLK


def _tc_counts_body(lab_ref, cnt_ref):
    i = pl.program_id(0)

    @pl.when(i == 0)
    def _():
        cnt_ref[...] = jnp.zeros((_CPAD, 1), jnp.float32)

    lab = lab_ref[0]
    oh = (jnp.broadcast_to(lab, (_CPAD, _BLK))
          == lax.broadcasted_iota(jnp.int32, (_CPAD, _BLK), 0))
    ones = jnp.ones((_BLK, 1), jnp.float32)
    cnt_ref[...] += lax.dot_general(oh.astype(jnp.float32), ones,
                                    (((1,), (0,)), ((), ())),
                                    preferred_element_type=jnp.float32)


_tc_counts = pl.pallas_call(
    _tc_counts_body,
    grid=(_NBLK,),
    in_specs=[pl.BlockSpec((1, 1, _BLK), lambda i: (i, 0, 0))],
    out_specs=pl.BlockSpec((_CPAD, 1), lambda i: (0, 0)),
    out_shape=jax.ShapeDtypeStruct((_CPAD, 1), jnp.float32),
)


def _tc_body(emb_ref, psums_ref, cnt_ref, proto_ref, init_ref,
             newp_ref, pred_ref, dist_ref, pnorm_s, ninit_s):
    i = pl.program_id(0)

    @pl.when(i == 0)
    def _():
        sums = psums_ref[0, :_C] + psums_ref[1, :_C]
        cnt = cnt_ref[...][:_C]
        cls_mean = sums / jnp.maximum(cnt, 1.0)
        present = cnt > 0.0
        initm = init_ref[...] > 0.0
        protos = proto_ref[...]
        ema = _MOM * protos + (1.0 - _MOM) * cls_mean
        upd = jnp.where(initm, ema, cls_mean)
        newp = jnp.where(present, upd, protos)
        newp_ref[...] = newp
        new_init = jnp.logical_or(initm, present)
        ninit_s[...] = new_init.astype(jnp.float32)

        pn = jnp.sqrt(jnp.sum(newp * newp, axis=1, keepdims=True))
        pnorm_s[...] = newp / jnp.maximum(pn, 1e-12)

    pnorm = pnorm_s[...]
    new_init = ninit_s[...] > 0.0

    e = emb_ref[...]
    en = jnp.sqrt(jnp.sum(e * e, axis=1, keepdims=True))
    en_inv = e / jnp.maximum(en, 1e-12)

    simsT = lax.dot_general(pnorm, en_inv, (((1,), (1,)), ((), ())),
                            preferred_element_type=jnp.float32)
    simsT = jnp.where(new_init, simsT, -jnp.inf)
    m = jnp.max(simsT, axis=0, keepdims=True)
    row = lax.broadcasted_iota(jnp.int32, simsT.shape, 0)
    pred = jnp.min(jnp.where(simsT == m, row, _C), axis=0, keepdims=True)
    pred_ref[0] = pred
    dist_ref[0] = 1.0 - m


_tc_predict = pl.pallas_call(
    _tc_body,
    grid=(_NMBLK,),
    in_specs=[
        pl.BlockSpec((_MBLK, _D), lambda i: (i, 0)),
        pl.BlockSpec((_NC, _CPAD, _D), lambda i: (0, 0, 0)),
        pl.BlockSpec((_CPAD, 1), lambda i: (0, 0)),
        pl.BlockSpec((_C, _D), lambda i: (0, 0)),
        pl.BlockSpec((_C, 1), lambda i: (0, 0)),
    ],
    out_specs=[
        pl.BlockSpec((_C, _D), lambda i: (0, 0)),
        pl.BlockSpec((1, 1, _MBLK), lambda i: (i, 0, 0)),
        pl.BlockSpec((1, 1, _MBLK), lambda i: (i, 0, 0)),
    ],
    out_shape=[
        jax.ShapeDtypeStruct((_C, _D), jnp.float32),
        jax.ShapeDtypeStruct((_NMBLK, 1, _MBLK), jnp.int32),
        jax.ShapeDtypeStruct((_NMBLK, 1, _MBLK), jnp.float32),
    ],
    scratch_shapes=[pltpu.VMEM((_C, _D), jnp.float32),
                    pltpu.VMEM((_C, 1), jnp.float32)],
)


def kernel(embeddings, labels, prototypes, initialized):
    zeros = jnp.zeros((_ZROWS, _D), jnp.float32)
    psums = _sc_segsum()(embeddings, labels, zeros)
    lab3 = labels.reshape(_NBLK, 1, _BLK)
    cnts = _tc_counts(lab3)
    init_col = initialized.astype(jnp.float32).reshape(_C, 1)
    newp, pred2d, dist2d = _tc_predict(embeddings, psums, cnts,
                                       prototypes, init_col)
    return newp, pred2d.reshape(_B), dist2d.reshape(_B)


# speedup vs baseline: 1.0545x; 1.0545x over previous
import functools

import jax
import jax.numpy as jnp
from jax import lax
from jax.experimental import pallas as pl
from jax.experimental.pallas import tpu as pltpu
from jax.experimental.pallas import tpu_sc as plsc

_B, _D, _C = 16384, 128, 100
_MOM = 0.9

_NC, _NS = 2, 16
_NW = _NC * _NS
_RPW = _B // _NW
_CHUNK = 128
_NCHUNK = _RPW // _CHUNK
_CPAD = 128
_ZROWS = _CPAD // _NS


def _sc_segsum_body(emb_hbm, lab_hbm, zeros_hbm, psums_hbm,
                    emb_v0, emb_v1, emb_v2, emb_v3,
                    idx_v0, idx_v1, idx_v2, idx_v3, sums_sh,
                    sem0, sem1, sem2, sem3, ssem):
    cid = lax.axis_index("c")
    sid = lax.axis_index("s")
    base = (cid * _NS + sid) * _RPW

    embs = (emb_v0, emb_v1, emb_v2, emb_v3)
    idxs = (idx_v0, idx_v1, idx_v2, idx_v3)
    sems = (sem0, sem1, sem2, sem3)

    descs = []
    for k in range(_NCHUNK):
        off = base + k * _CHUNK
        descs.append((
            pltpu.async_copy(lab_hbm.at[pl.ds(off, _CHUNK)], idxs[k], sems[k]),
            pltpu.async_copy(emb_hbm.at[pl.ds(off, _CHUNK)], embs[k], sems[k]),
        ))

    pltpu.sync_copy(zeros_hbm, sums_sh.at[pl.ds(sid * _ZROWS, _ZROWS)])
    plsc.subcore_barrier()

    scat = []
    for k in range(_NCHUNK):
        d1, d2 = descs[k]
        d1.wait()
        d2.wait()
        scat.append(pltpu.async_copy(embs[k], sums_sh.at[idxs[k]], ssem,
                                     add=True))
    for d in scat:
        d.wait()
    plsc.subcore_barrier()

    r0 = sid * _ZROWS
    pltpu.sync_copy(sums_sh.at[pl.ds(r0, _ZROWS)],
                    psums_hbm.at[cid, pl.ds(r0, _ZROWS)])


@functools.cache
def _sc_segsum():
    return pl.kernel(
        _sc_segsum_body,
        out_type=jax.ShapeDtypeStruct((_NC, _CPAD, _D), jnp.float32),
        mesh=plsc.VectorSubcoreMesh(core_axis_name="c", subcore_axis_name="s",
                                    num_cores=_NC, num_subcores=_NS),
        scratch_types=(
            [pltpu.VMEM((_CHUNK, _D), jnp.float32)] * _NCHUNK
            + [pltpu.VMEM((_CHUNK,), jnp.int32)] * _NCHUNK
            + [pltpu.VMEM_SHARED((_CPAD, _D), jnp.float32)]
            + [pltpu.SemaphoreType.DMA] * (_NCHUNK + 1)
        ),
    )


_BLK = 2048
_NBLK = _B // _BLK
_MBLK = 4096
_NMBLK = _B // _MBLK


def _tc_counts_body(lab_ref, cnt_ref):
    i = pl.program_id(0)

    @pl.when(i == 0)
    def _():
        cnt_ref[...] = jnp.zeros((_CPAD, 1), jnp.float32)

    lab = lab_ref[0]
    oh = (jnp.broadcast_to(lab, (_CPAD, _BLK))
          == lax.broadcasted_iota(jnp.int32, (_CPAD, _BLK), 0))
    ones = jnp.ones((_BLK, 1), jnp.float32)
    cnt_ref[...] += lax.dot_general(oh.astype(jnp.float32), ones,
                                    (((1,), (0,)), ((), ())),
                                    preferred_element_type=jnp.float32)


_tc_counts = pl.pallas_call(
    _tc_counts_body,
    grid=(_NBLK,),
    in_specs=[pl.BlockSpec((1, 1, _BLK), lambda i: (i, 0, 0))],
    out_specs=pl.BlockSpec((_CPAD, 1), lambda i: (0, 0)),
    out_shape=jax.ShapeDtypeStruct((_CPAD, 1), jnp.float32),
)


def _tc_body(emb_ref, psums_ref, cnt_ref, proto_ref, init_ref,
             newp_ref, pred_ref, dist_ref):
    sums = psums_ref[0, :_C] + psums_ref[1, :_C]
    cnt = cnt_ref[...][:_C]
    cls_mean = sums / jnp.maximum(cnt, 1.0)
    present = cnt > 0.0
    initm = init_ref[...] > 0.0
    protos = proto_ref[...]
    ema = _MOM * protos + (1.0 - _MOM) * cls_mean
    upd = jnp.where(initm, ema, cls_mean)
    newp = jnp.where(present, upd, protos)
    newp_ref[...] = newp
    new_init = jnp.logical_or(initm, present)

    pn = jnp.sqrt(jnp.sum(newp * newp, axis=1, keepdims=True))
    pnorm = newp / jnp.maximum(pn, 1e-12)

    e = emb_ref[...]
    sq_t = lax.dot_general(jnp.ones((1, _D), jnp.float32), e * e,
                           (((1,), (1,)), ((), ())),
                           preferred_element_type=jnp.float32)
    en_t = jnp.sqrt(sq_t)

    rawT = lax.dot_general(pnorm, e, (((1,), (1,)), ((), ())),
                           preferred_element_type=jnp.float32)
    rawT = jnp.where(new_init, rawT, -jnp.inf)
    m = jnp.max(rawT, axis=0, keepdims=True)
    row = lax.broadcasted_iota(jnp.int32, rawT.shape, 0)
    pred = jnp.min(jnp.where(rawT == m, row, _C), axis=0, keepdims=True)
    pred_ref[0] = pred
    dist_ref[0] = 1.0 - m / jnp.maximum(en_t, 1e-12)


_tc_predict = pl.pallas_call(
    _tc_body,
    grid=(_NMBLK,),
    in_specs=[
        pl.BlockSpec((_MBLK, _D), lambda i: (i, 0)),
        pl.BlockSpec((_NC, _CPAD, _D), lambda i: (0, 0, 0)),
        pl.BlockSpec((_CPAD, 1), lambda i: (0, 0)),
        pl.BlockSpec((_C, _D), lambda i: (0, 0)),
        pl.BlockSpec((_C, 1), lambda i: (0, 0)),
    ],
    out_specs=[
        pl.BlockSpec((_C, _D), lambda i: (0, 0)),
        pl.BlockSpec((1, 1, _MBLK), lambda i: (i, 0, 0)),
        pl.BlockSpec((1, 1, _MBLK), lambda i: (i, 0, 0)),
    ],
    out_shape=[
        jax.ShapeDtypeStruct((_C, _D), jnp.float32),
        jax.ShapeDtypeStruct((_NMBLK, 1, _MBLK), jnp.int32),
        jax.ShapeDtypeStruct((_NMBLK, 1, _MBLK), jnp.float32),
    ],
)


def kernel(embeddings, labels, prototypes, initialized):
    zeros = jnp.zeros((_ZROWS, _D), jnp.float32)
    psums = _sc_segsum()(embeddings, labels, zeros)
    lab3 = labels.reshape(_NBLK, 1, _BLK)
    cnts = _tc_counts(lab3)
    init_col = initialized.astype(jnp.float32).reshape(_C, 1)
    newp, pred2d, dist2d = _tc_predict(embeddings, psums, cnts,
                                       prototypes, init_col)
    return newp, pred2d.reshape(_B), dist2d.reshape(_B)
